# Initial kernel scaffold; baseline (speedup 1.0000x reference)
import jax
import jax.numpy as jnp
from jax.experimental import pallas as pl


def kernel(x, duration, max_len):
    raise NotImplementedError("write your pallas kernel here")


# speedup vs baseline: 4.3977x; 4.3977x over previous
import functools

import jax
import jax.numpy as jnp
from jax import lax
from jax.experimental import pallas as pl
from jax.experimental.pallas import tpu as pltpu
from jax.experimental.pallas import tpu_sc as plsc

B, L, D = 8, 512, 256
ML = 3584
NW = 32
WPR = NW // B
POS_PW = ML // WPR
CHUNK = 128
NCH = POS_PW // CHUNK
ZROW = B * L


def _lr_body(xpad_hbm, dur_hbm, out_hbm, mel_hbm,
             d_v, s_v, ix_v, mel_v, buf0, buf1, sem0, sem1):
    cid = lax.axis_index("c")
    sid = lax.axis_index("s")
    wid = cid * 16 + sid
    row = wid // WPR
    quarter = wid % WPR

    pltpu.sync_copy(dur_hbm.at[row], d_v)

    lane = lax.iota(jnp.int32, 16)

    def _zero(j, _):
        s_v[pl.ds(j * 16, 16)] = jnp.zeros((16,), jnp.int32)
        return 0
    lax.fori_loop(0, ML // 16, _zero, 0)

    def _scan(j, carry):
        d = d_v[pl.ds(j * 16, 16)]
        cs = plsc.cumsum(d) + carry
        start = cs - d
        tok = j * 16 + lane
        m = (d > 0) & (start < ML)
        plsc.store_scatter(s_v, [start], tok, mask=m)
        return jnp.max(cs)
    total = lax.fori_loop(0, L // 16, _scan, jnp.int32(0))

    rbase = row * L

    def _cmax(j, carry):
        v = s_v[pl.ds(j * 16, 16)]
        cm = jnp.maximum(plsc.cummax(v), carry)
        pos = j * 16 + lane
        src = jnp.where(pos < total, rbase + cm, ZROW)
        ix_v[pl.ds(j * 16, 16)] = src
        return jnp.max(cm)
    lax.fori_loop(0, ML // 16, _cmax, jnp.int32(0))

    @pl.when(quarter == 0)
    def _():
        mel_v[...] = jnp.full((16,), total, jnp.int32)
        pltpu.sync_copy(mel_v, mel_hbm.at[row])

    qoff = quarter * POS_PW
    obase = row * ML + qoff
    bufs = (buf0, buf1)
    sems = (sem0, sem1)
    cps = [None] * NCH
    for c in range(NCH):
        cps[c] = pltpu.async_copy(
            xpad_hbm.at[ix_v.at[pl.ds(qoff + c * CHUNK, CHUNK)]],
            bufs[c % 2], sems[c % 2])
        if c > 0:
            cps[c - 1].wait()
            pltpu.sync_copy(bufs[(c - 1) % 2],
                            out_hbm.at[pl.ds(obase + (c - 1) * CHUNK, CHUNK)])
    cps[NCH - 1].wait()
    pltpu.sync_copy(bufs[(NCH - 1) % 2],
                    out_hbm.at[pl.ds(obase + (NCH - 1) * CHUNK, CHUNK)])


_lr_call = functools.partial(
    pl.kernel,
    out_type=[
        jax.ShapeDtypeStruct((B * ML, D), jnp.float32),
        jax.ShapeDtypeStruct((B, 16), jnp.int32),
    ],
    mesh=plsc.VectorSubcoreMesh(core_axis_name="c", subcore_axis_name="s"),
    compiler_params=pltpu.CompilerParams(needs_layout_passes=False),
    scratch_types=[
        pltpu.VMEM((L,), jnp.int32),
        pltpu.VMEM((ML,), jnp.int32),
        pltpu.VMEM((ML,), jnp.int32),
        pltpu.VMEM((16,), jnp.int32),
        pltpu.VMEM((CHUNK, D), jnp.float32),
        pltpu.VMEM((CHUNK, D), jnp.float32),
        pltpu.SemaphoreType.DMA,
        pltpu.SemaphoreType.DMA,
    ],
)(_lr_body)


def kernel(x, duration, max_len):
    xpad = jnp.concatenate(
        [x.reshape(B * L, D), jnp.zeros((8, D), x.dtype)], axis=0)
    out_flat, mel = _lr_call(xpad, duration)
    return out_flat.reshape(B, ML, D), mel[:, 0]
